# baseline (speedup 1.0000x reference)
import functools

import jax
import jax.numpy as jnp
from jax import lax
from jax.experimental import pallas as pl
from jax.experimental.pallas import tpu as pltpu
from jax.experimental.pallas import tpu_sc as plsc

N_TOKENS = 32
WINDOW = 12
LANES = 16
N_WORKERS = 32


def _sax_body(n_rows, t_len, x_hbm, bp_hbm, out_hbm,
              buf0, buf1, wsum, tok0, tok1, bpv, tbl,
              sin0, sin1, so0, so1):
    n_win = t_len // WINDOW
    n_grp = (n_win + LANES - 1) // LANES
    n_full = n_win // LANES
    rem = n_win - n_full * LANES
    rows_per = n_rows // N_WORKERS

    cid = lax.axis_index("c")
    sid = lax.axis_index("s")
    wid = sid * 2 + cid
    base = wid * rows_per

    zf = jnp.zeros((LANES,), jnp.float32)
    lanes = lax.iota(jnp.int32, LANES)
    lanes12 = lanes * WINDOW

    pltpu.make_async_copy(x_hbm.at[base, 0], buf0, sin0).start()
    pltpu.sync_copy(bp_hbm, bpv)

    def process(buf, tokbuf):
        def g_body(g, carry):
            sacc, ssum = carry
            gbase = g * (LANES * WINDOW)
            wacc = zf
            for k in range(WINDOW):
                v = plsc.load_gather(buf, [gbase + k + lanes12])
                wacc = wacc + v
                sacc = sacc + v * v
            wsum[pl.ds(g * LANES, LANES)] = wacc
            return (sacc, ssum + wacc)

        sacc, ssum = lax.fori_loop(0, n_full, g_body, (zf, zf))

        if rem:
            gbase = n_full * (LANES * WINDOW)
            valid = lanes < rem
            wacc = zf
            for k in range(WINDOW):
                idx = jnp.minimum(gbase + k + lanes12,
                                  jnp.int32(t_len - 1))
                v = jnp.where(valid, plsc.load_gather(buf, [idx]), 0.0)
                wacc = wacc + v
                sacc = sacc + v * v
            wsum[pl.ds(n_full * LANES, LANES)] = wacc
            ssum = ssum + wacc

        s_tot = jnp.sum(ssum)
        q_tot = jnp.sum(sacc)

        inv_t = jnp.float32(1.0 / t_len)
        inv_t1 = jnp.float32(1.0 / (t_len - 1))
        var = jnp.maximum((q_tot - s_tot * s_tot * inv_t) * inv_t1,
                          jnp.float32(1e-30))
        var_v = jnp.broadcast_to(var, (LANES,))
        iv = plsc.bitcast(var_v, jnp.int32)
        iv = jnp.int32(0x5F3759DF) - (iv >> 1)
        y = plsc.bitcast(iv, jnp.float32)
        for _ in range(3):
            y = y * (1.5 - 0.5 * var_v * y * y)
        std_v = var_v * y
        scale_v = jnp.float32(WINDOW) * (std_v + 1e-8)
        off_v = jnp.broadcast_to(s_tot * jnp.float32(WINDOW / t_len), (LANES,))
        tbl[pl.ds(0, LANES)] = bpv[pl.ds(0, LANES)] * scale_v + off_v
        tbl[pl.ds(LANES, LANES)] = bpv[pl.ds(LANES, LANES)] * scale_v + off_v

        def search(w):
            pos = jnp.zeros((LANES,), jnp.int32)
            for s in (16, 8, 4, 2, 1):
                t = plsc.load_gather(tbl, [pos + (s - 1)])
                pos = jnp.where(t < w, pos + s, pos)
            return pos

        def t_body(g, _):
            gb = g * LANES
            tokbuf[pl.ds(gb, LANES)] = search(wsum[pl.ds(gb, LANES)])
            return 0

        lax.fori_loop(0, n_full, t_body, 0)
        if rem:
            pos = search(wsum[pl.ds(n_full * LANES, LANES)])
            plsc.store_scatter(tokbuf, [n_full * LANES + lanes], pos,
                               mask=lanes < rem)

    def outer(i, _):
        for ph in range(2):
            buf, sin = (buf0, sin0) if ph == 0 else (buf1, sin1)
            nbuf, nsin = (buf1, sin1) if ph == 0 else (buf0, sin0)
            tokbuf, so = (tok0, so0) if ph == 0 else (tok1, so1)
            r = i * 2 + ph
            row = base + r

            pltpu.make_async_copy(x_hbm.at[row, 0], buf, sin).wait()

            @pl.when(r + 1 < rows_per)
            def _():
                pltpu.make_async_copy(x_hbm.at[row + 1, 0], nbuf, nsin).start()

            @pl.when(r >= 2)
            def _():
                pltpu.make_async_copy(tokbuf, out_hbm.at[row - 2], so).wait()

            process(buf, tokbuf)
            pltpu.make_async_copy(tokbuf, out_hbm.at[row], so).start()
        return 0

    lax.fori_loop(0, rows_per // 2, outer, 0)
    pltpu.make_async_copy(tok0, out_hbm.at[base + rows_per - 2], so0).wait()
    pltpu.make_async_copy(tok1, out_hbm.at[base + rows_per - 1], so1).wait()


def _gaussian_breakpoints_padded():
    probs = jnp.linspace(0.0, 1.0, N_TOKENS + 1)[1:-1]
    bp = jnp.sqrt(2.0) * jax.scipy.special.erfinv(2.0 * probs - 1.0)
    return jnp.concatenate([bp, jnp.array([jnp.inf], jnp.float32)])


def kernel(x):
    n_rows, _, t_len = x.shape
    n_win = t_len // WINDOW
    n_grp = (n_win + LANES - 1) // LANES
    pad_w = n_grp * LANES

    bp = _gaussian_breakpoints_padded()

    mesh = plsc.VectorSubcoreMesh(core_axis_name="c", subcore_axis_name="s")
    run = pl.kernel(
        functools.partial(_sax_body, n_rows, t_len),
        out_type=jax.ShapeDtypeStruct((n_rows, n_win), jnp.int32),
        mesh=mesh,
        compiler_params=pltpu.CompilerParams(needs_layout_passes=False,
                                             use_tc_tiling_on_sc=True),
        scratch_types=[
            pltpu.VMEM((t_len,), jnp.float32),
            pltpu.VMEM((t_len,), jnp.float32),
            pltpu.VMEM((pad_w,), jnp.float32),
            pltpu.VMEM((n_win,), jnp.int32),
            pltpu.VMEM((n_win,), jnp.int32),
            pltpu.VMEM((N_TOKENS,), jnp.float32),
            pltpu.VMEM((N_TOKENS,), jnp.float32),
            pltpu.SemaphoreType.DMA,
            pltpu.SemaphoreType.DMA,
            pltpu.SemaphoreType.DMA,
            pltpu.SemaphoreType.DMA,
        ],
    )
    return run(x, bp)


# speedup vs baseline: 1.5473x; 1.5473x over previous
import functools

import jax
import jax.numpy as jnp
from jax import lax
from jax.experimental import pallas as pl
from jax.experimental.pallas import tpu as pltpu
from jax.experimental.pallas import tpu_sc as plsc

N_TOKENS = 32
WINDOW = 12
LANES = 16
N_WORKERS = 32


def _sax_body(n_rows, t_len, x_hbm, bp_hbm, out_hbm,
              buf0, buf1, wsum, tok0, tok1, bpv, tbl,
              sin0, sin1, so0, so1):
    n_win = t_len // WINDOW
    n_grp = (n_win + LANES - 1) // LANES
    n_full = n_win // LANES
    rem = n_win - n_full * LANES
    rows_per = n_rows // N_WORKERS

    cid = lax.axis_index("c")
    sid = lax.axis_index("s")
    wid = sid * 2 + cid
    base = wid * rows_per

    zf = jnp.zeros((LANES,), jnp.float32)
    lanes = lax.iota(jnp.int32, LANES)
    lanes12 = lanes * WINDOW

    pltpu.make_async_copy(x_hbm.at[base], buf0, sin0).start()
    pltpu.sync_copy(bp_hbm, bpv)

    def tree_sum(vals):
        vals = list(vals)
        while len(vals) > 1:
            nxt = [a + b for a, b in zip(vals[::2], vals[1::2])]
            if len(vals) % 2:
                nxt.append(vals[-1])
            vals = nxt
        return vals[0]

    def process(buf, tokbuf):
        def one_group(g, sacc, ssum):
            gbase = g * (LANES * WINDOW)
            vs = [plsc.load_gather(buf, [gbase + k + lanes12])
                  for k in range(WINDOW)]
            wacc = tree_sum(vs)
            sacc = sacc + tree_sum([v * v for v in vs])
            wsum[pl.ds(g * LANES, LANES)] = wacc
            return sacc, ssum + wacc

        def g_body(h, carry):
            sacc, ssum = carry
            sacc, ssum = one_group(2 * h, sacc, ssum)
            return one_group(2 * h + 1, sacc, ssum)

        sacc, ssum = lax.fori_loop(0, n_full // 2, g_body, (zf, zf))
        if n_full % 2:
            sacc, ssum = one_group(n_full - 1, sacc, ssum)

        if rem:
            gbase = n_full * (LANES * WINDOW)
            valid = lanes < rem
            wacc = zf
            for k in range(WINDOW):
                idx = jnp.minimum(gbase + k + lanes12,
                                  jnp.int32(t_len - 1))
                v = jnp.where(valid, plsc.load_gather(buf, [idx]), 0.0)
                wacc = wacc + v
                sacc = sacc + v * v
            wsum[pl.ds(n_full * LANES, LANES)] = wacc
            ssum = ssum + wacc

        s_tot = jnp.sum(ssum)
        q_tot = jnp.sum(sacc)

        inv_t = jnp.float32(1.0 / t_len)
        inv_t1 = jnp.float32(1.0 / (t_len - 1))
        var = jnp.maximum((q_tot - s_tot * s_tot * inv_t) * inv_t1,
                          jnp.float32(1e-30))
        var_v = jnp.broadcast_to(var, (LANES,))
        iv = plsc.bitcast(var_v, jnp.int32)
        iv = jnp.int32(0x5F3759DF) - (iv >> 1)
        y = plsc.bitcast(iv, jnp.float32)
        for _ in range(3):
            y = y * (1.5 - 0.5 * var_v * y * y)
        std_v = var_v * y
        scale_v = jnp.float32(WINDOW) * (std_v + 1e-8)
        off_v = jnp.broadcast_to(s_tot * jnp.float32(WINDOW / t_len), (LANES,))
        tbl[pl.ds(0, LANES)] = bpv[pl.ds(0, LANES)] * scale_v + off_v
        tbl[pl.ds(LANES, LANES)] = bpv[pl.ds(LANES, LANES)] * scale_v + off_v

        def search(w):
            pos = jnp.zeros((LANES,), jnp.int32)
            for s in (16, 8, 4, 2, 1):
                t = plsc.load_gather(tbl, [pos + (s - 1)])
                pos = jnp.where(t < w, pos + s, pos)
            return pos

        def t_body(h, _):
            for g in (2 * h, 2 * h + 1):
                gb = g * LANES
                tokbuf[pl.ds(gb, LANES)] = search(wsum[pl.ds(gb, LANES)])
            return 0

        lax.fori_loop(0, n_full // 2, t_body, 0)
        if n_full % 2:
            gb = (n_full - 1) * LANES
            tokbuf[pl.ds(gb, LANES)] = search(wsum[pl.ds(gb, LANES)])
        if rem:
            pos = search(wsum[pl.ds(n_full * LANES, LANES)])
            plsc.store_scatter(tokbuf, [n_full * LANES + lanes], pos,
                               mask=lanes < rem)

    def outer(i, _):
        for ph in range(2):
            buf, sin = (buf0, sin0) if ph == 0 else (buf1, sin1)
            nbuf, nsin = (buf1, sin1) if ph == 0 else (buf0, sin0)
            tokbuf, so = (tok0, so0) if ph == 0 else (tok1, so1)
            r = i * 2 + ph
            row = base + r

            pltpu.make_async_copy(x_hbm.at[row], buf, sin).wait()

            @pl.when(r + 1 < rows_per)
            def _():
                pltpu.make_async_copy(x_hbm.at[row + 1], nbuf, nsin).start()

            @pl.when(r >= 2)
            def _():
                pltpu.make_async_copy(tokbuf, out_hbm.at[row - 2], so).wait()

            process(buf, tokbuf)
            pltpu.make_async_copy(tokbuf, out_hbm.at[row], so).start()
        return 0

    lax.fori_loop(0, rows_per // 2, outer, 0)
    pltpu.make_async_copy(tok0, out_hbm.at[base + rows_per - 2], so0).wait()
    pltpu.make_async_copy(tok1, out_hbm.at[base + rows_per - 1], so1).wait()


def _gaussian_breakpoints_padded():
    probs = jnp.linspace(0.0, 1.0, N_TOKENS + 1)[1:-1]
    bp = jnp.sqrt(2.0) * jax.scipy.special.erfinv(2.0 * probs - 1.0)
    return jnp.concatenate([bp, jnp.array([jnp.inf], jnp.float32)])


def kernel(x):
    n_rows, _, t_len = x.shape
    n_win = t_len // WINDOW
    n_grp = (n_win + LANES - 1) // LANES
    pad_w = n_grp * LANES

    x2 = x.reshape(n_rows, t_len)
    bp = _gaussian_breakpoints_padded()

    mesh = plsc.VectorSubcoreMesh(core_axis_name="c", subcore_axis_name="s")
    run = pl.kernel(
        functools.partial(_sax_body, n_rows, t_len),
        out_type=jax.ShapeDtypeStruct((n_rows, n_win), jnp.int32),
        mesh=mesh,
        compiler_params=pltpu.CompilerParams(needs_layout_passes=False),
        scratch_types=[
            pltpu.VMEM((t_len,), jnp.float32),
            pltpu.VMEM((t_len,), jnp.float32),
            pltpu.VMEM((pad_w,), jnp.float32),
            pltpu.VMEM((n_win,), jnp.int32),
            pltpu.VMEM((n_win,), jnp.int32),
            pltpu.VMEM((N_TOKENS,), jnp.float32),
            pltpu.VMEM((N_TOKENS,), jnp.float32),
            pltpu.SemaphoreType.DMA,
            pltpu.SemaphoreType.DMA,
            pltpu.SemaphoreType.DMA,
            pltpu.SemaphoreType.DMA,
        ],
    )
    return run(x2, bp)


# speedup vs baseline: 2.6516x; 1.7137x over previous
import functools

import jax
import jax.numpy as jnp
from jax import lax
from jax.experimental import pallas as pl
from jax.experimental.pallas import tpu as pltpu
from jax.experimental.pallas import tpu_sc as plsc

N_TOKENS = 32
WINDOW = 12
LANES = 16
N_WORKERS = 32


def _sax_body(n_rows, t_len, x_hbm, bp_hbm, out_hbm,
              buf0, buf1, wsum, tok0, tok1, bpv, tbl,
              sin0, sin1, so0, so1):
    n_win = t_len // WINDOW
    n_grp = (n_win + LANES - 1) // LANES
    n_full = n_win // LANES
    rem = n_win - n_full * LANES
    rows_per = n_rows // N_WORKERS

    cid = lax.axis_index("c")
    sid = lax.axis_index("s")
    wid = sid * 2 + cid
    base = wid * rows_per

    zf = jnp.zeros((LANES,), jnp.float32)
    lanes = lax.iota(jnp.int32, LANES)
    lanes12 = lanes * WINDOW

    pltpu.make_async_copy(x_hbm.at[base], buf0, sin0).start()
    pltpu.sync_copy(bp_hbm, bpv)

    def tree_sum(vals):
        vals = list(vals)
        while len(vals) > 1:
            nxt = [a + b for a, b in zip(vals[::2], vals[1::2])]
            if len(vals) % 2:
                nxt.append(vals[-1])
            vals = nxt
        return vals[0]

    def process(buf, tokbuf):
        def one_group(g, sacc, ssum):
            gbase = g * (LANES * WINDOW)
            vs = [plsc.load_gather(buf, [gbase + k + lanes12])
                  for k in range(WINDOW)]
            wacc = tree_sum(vs)
            sacc = sacc + tree_sum([v * v for v in vs])
            wsum[pl.ds(g * LANES, LANES)] = wacc
            return sacc, ssum + wacc

        @plsc.parallel_loop(0, n_full, unroll=2, carry=(zf, zf))
        def p1_carry(g, carry):
            return one_group(g, *carry)

        sacc, ssum = p1_carry

        if rem:
            gbase = n_full * (LANES * WINDOW)
            valid = lanes < rem
            wacc = zf
            for k in range(WINDOW):
                idx = jnp.minimum(gbase + k + lanes12,
                                  jnp.int32(t_len - 1))
                v = jnp.where(valid, plsc.load_gather(buf, [idx]), 0.0)
                wacc = wacc + v
                sacc = sacc + v * v
            wsum[pl.ds(n_full * LANES, LANES)] = wacc
            ssum = ssum + wacc

        s_tot = jnp.sum(ssum)
        q_tot = jnp.sum(sacc)

        inv_t = jnp.float32(1.0 / t_len)
        inv_t1 = jnp.float32(1.0 / (t_len - 1))
        var = jnp.maximum((q_tot - s_tot * s_tot * inv_t) * inv_t1,
                          jnp.float32(1e-30))
        var_v = jnp.broadcast_to(var, (LANES,))
        iv = plsc.bitcast(var_v, jnp.int32)
        iv = jnp.int32(0x5F3759DF) - (iv >> 1)
        y = plsc.bitcast(iv, jnp.float32)
        for _ in range(3):
            y = y * (1.5 - 0.5 * var_v * y * y)
        std_v = var_v * y
        scale_v = jnp.float32(WINDOW) * (std_v + 1e-8)
        off_v = jnp.broadcast_to(s_tot * jnp.float32(WINDOW / t_len), (LANES,))
        tbl[pl.ds(0, LANES)] = bpv[pl.ds(0, LANES)] * scale_v + off_v
        tbl[pl.ds(LANES, LANES)] = bpv[pl.ds(LANES, LANES)] * scale_v + off_v

        def search(w):
            pos = jnp.zeros((LANES,), jnp.int32)
            for s in (16, 8, 4, 2, 1):
                t = plsc.load_gather(tbl, [pos + (s - 1)])
                pos = jnp.where(t < w, pos + s, pos)
            return pos

        @plsc.parallel_loop(0, n_full, unroll=4)
        def _(g):
            gb = g * LANES
            tokbuf[pl.ds(gb, LANES)] = search(wsum[pl.ds(gb, LANES)])
        if rem:
            pos = search(wsum[pl.ds(n_full * LANES, LANES)])
            plsc.store_scatter(tokbuf, [n_full * LANES + lanes], pos,
                               mask=lanes < rem)

    def outer(i, _):
        for ph in range(2):
            buf, sin = (buf0, sin0) if ph == 0 else (buf1, sin1)
            nbuf, nsin = (buf1, sin1) if ph == 0 else (buf0, sin0)
            tokbuf, so = (tok0, so0) if ph == 0 else (tok1, so1)
            r = i * 2 + ph
            row = base + r

            pltpu.make_async_copy(x_hbm.at[row], buf, sin).wait()

            @pl.when(r + 1 < rows_per)
            def _():
                pltpu.make_async_copy(x_hbm.at[row + 1], nbuf, nsin).start()

            @pl.when(r >= 2)
            def _():
                pltpu.make_async_copy(tokbuf, out_hbm.at[row - 2], so).wait()

            process(buf, tokbuf)
            pltpu.make_async_copy(tokbuf, out_hbm.at[row], so).start()
        return 0

    lax.fori_loop(0, rows_per // 2, outer, 0)
    pltpu.make_async_copy(tok0, out_hbm.at[base + rows_per - 2], so0).wait()
    pltpu.make_async_copy(tok1, out_hbm.at[base + rows_per - 1], so1).wait()


def _gaussian_breakpoints_padded():
    probs = jnp.linspace(0.0, 1.0, N_TOKENS + 1)[1:-1]
    bp = jnp.sqrt(2.0) * jax.scipy.special.erfinv(2.0 * probs - 1.0)
    return jnp.concatenate([bp, jnp.array([jnp.inf], jnp.float32)])


def kernel(x):
    n_rows, _, t_len = x.shape
    n_win = t_len // WINDOW
    n_grp = (n_win + LANES - 1) // LANES
    pad_w = n_grp * LANES

    x2 = x.reshape(n_rows, t_len)
    bp = _gaussian_breakpoints_padded()

    mesh = plsc.VectorSubcoreMesh(core_axis_name="c", subcore_axis_name="s")
    run = pl.kernel(
        functools.partial(_sax_body, n_rows, t_len),
        out_type=jax.ShapeDtypeStruct((n_rows, n_win), jnp.int32),
        mesh=mesh,
        compiler_params=pltpu.CompilerParams(needs_layout_passes=False),
        scratch_types=[
            pltpu.VMEM((t_len,), jnp.float32),
            pltpu.VMEM((t_len,), jnp.float32),
            pltpu.VMEM((pad_w,), jnp.float32),
            pltpu.VMEM((n_win,), jnp.int32),
            pltpu.VMEM((n_win,), jnp.int32),
            pltpu.VMEM((N_TOKENS,), jnp.float32),
            pltpu.VMEM((N_TOKENS,), jnp.float32),
            pltpu.SemaphoreType.DMA,
            pltpu.SemaphoreType.DMA,
            pltpu.SemaphoreType.DMA,
            pltpu.SemaphoreType.DMA,
        ],
    )
    return run(x2, bp)
